# baseline (speedup 1.0000x reference)
import functools

import jax
import jax.numpy as jnp
import numpy as np
from jax import lax
from jax.experimental import pallas as pl
from jax.experimental.pallas import tpu as pltpu
from jax.experimental.pallas import tpu_sc as plsc

_B, _L, _DIN, _DOUT, _E = 1024, 50, 768, 64, 8
_NTOK = _B * _L

_NC, _NS = 2, 16
_NW = _NC * _NS
_BPW = _NTOK // _NW
_C = 80
_K = _BPW // _C
_NBLK = 1000000 // 8


_RC = 160


def _gather_body(tab_hbm, idx_hbm, out_hbm, idx_v, rows_v, sem):
    wid = lax.axis_index("s") * _NC + lax.axis_index("c")
    base = wid * _BPW
    pltpu.sync_copy(idx_hbm.at[wid], idx_v)
    lanes = jnp.arange(16, dtype=jnp.int32)

    def big(bc, carry):
        def group(g, carry2):
            v16 = idx_v[pl.ds((bc * 10 + g) * 16, 16)]
            copies = []
            for j in range(16):
                i = jnp.sum(jnp.where(lanes == j, v16, 0))
                copies.append(
                    pltpu.async_copy(
                        tab_hbm.at[pl.ds(i, 1)],
                        rows_v.at[pl.ds(g * 16 + j, 1)],
                        sem,
                    )
                )
            for cp in copies:
                cp.wait()
            return carry2

        lax.fori_loop(0, _RC // 16, group, 0)
        pltpu.sync_copy(rows_v, out_hbm.at[pl.ds(base + bc * _RC, _RC)])
        return carry

    lax.fori_loop(0, _BPW // _RC, big, 0)


def _sc_gather(table, idx2):
    mesh = plsc.VectorSubcoreMesh(core_axis_name="c", subcore_axis_name="s")
    return pl.kernel(
        _gather_body,
        mesh=mesh,
        out_type=jax.ShapeDtypeStruct((_NTOK, _DOUT), jnp.float32),
        scratch_types=[
            pltpu.VMEM((_BPW,), jnp.int32),
            pltpu.VMEM((_RC, _DOUT), jnp.float32),
            pltpu.SemaphoreType.DMA,
        ],
        compiler_params=pltpu.CompilerParams(needs_layout_passes=False),
    )(table, idx2)


_T = 1024


def _moe_body(x_ref, wg_ref, we_ref, sel_ref, bg_ref, emb_ref, o_ref):
    x = x_ref[...]
    logits = jnp.dot(x, wg_ref[...], preferred_element_type=jnp.float32)
    g = jax.nn.softmax(logits, axis=-1)
    y = jnp.dot(
        x.astype(jnp.bfloat16), we_ref[...], preferred_element_type=jnp.float32
    )
    grep = jnp.dot(g, sel_ref[...], preferred_element_type=jnp.float32)
    z = (y * grep).reshape(_T, 4, 128)
    s = jnp.sum(z, axis=1)
    acc = s[:, :_DOUT] + s[:, _DOUT:]
    acc = acc + jnp.dot(g, bg_ref[...], preferred_element_type=jnp.float32)
    o_ref[...] = acc + emb_ref[...]


def _tc_moe(x, wg, we_cat, sel, be, emb):
    return pl.pallas_call(
        _moe_body,
        grid=(_NTOK // _T,),
        in_specs=[
            pl.BlockSpec((_T, _DIN), lambda i: (i, 0)),
            pl.BlockSpec((_DIN, _E), lambda i: (0, 0)),
            pl.BlockSpec((_DIN, _E * _DOUT), lambda i: (0, 0)),
            pl.BlockSpec((_E, _E * _DOUT), lambda i: (0, 0)),
            pl.BlockSpec((_E, _DOUT), lambda i: (0, 0)),
            pl.BlockSpec((_T, _DOUT), lambda i: (i, 0)),
        ],
        out_specs=pl.BlockSpec((_T, _DOUT), lambda i: (i, 0)),
        out_shape=jax.ShapeDtypeStruct((_NTOK, _DOUT), jnp.float32),
    )(x, wg, we_cat, sel, be, emb)


def kernel(item_feat, item_id, table, W_e, b_e, W_g):
    x = item_feat.transpose(1, 0, 2).reshape(_NTOK, _DIN)
    idx2 = item_id.T.reshape(_NW, _BPW).astype(jnp.int32)
    emb = _sc_gather(table, idx2)
    we_cat = jnp.transpose(W_e, (1, 0, 2)).reshape(_DIN, _E * _DOUT)
    we_cat = we_cat.astype(jnp.bfloat16)
    sel = jnp.asarray(np.repeat(np.eye(_E, dtype=np.float32), _DOUT, axis=1))
    out = _tc_moe(x, W_g, we_cat, sel, b_e, emb)
    return out.reshape(_L, _B, _DOUT).transpose(1, 0, 2)


# speedup vs baseline: 1.1097x; 1.1097x over previous
import functools

import jax
import jax.numpy as jnp
import numpy as np
from jax import lax
from jax.experimental import pallas as pl
from jax.experimental.pallas import tpu as pltpu
from jax.experimental.pallas import tpu_sc as plsc

_B, _L, _DIN, _DOUT, _E = 1024, 50, 768, 64, 8
_NTOK = _B * _L

_NC, _NS = 2, 16
_NW = _NC * _NS
_BPW = _NTOK // _NW
_C = 80
_K = _BPW // _C
_NBLK = 1000000 // 8


_RC = 160


def _gather_body(tab_hbm, idx_hbm, out_hbm, idx_v, rows_v, sem):
    wid = lax.axis_index("s") * _NC + lax.axis_index("c")
    base = wid * _BPW
    pltpu.sync_copy(idx_hbm.at[wid], idx_v)
    lanes = jnp.arange(16, dtype=jnp.int32)

    def big(bc, carry):
        def group(g, carry2):
            v16 = idx_v[pl.ds((bc * 10 + g) * 16, 16)]
            copies = []
            for j in range(16):
                i = jnp.sum(jnp.where(lanes == j, v16, 0))
                copies.append(
                    pltpu.async_copy(
                        tab_hbm.at[pl.ds(i, 1)],
                        rows_v.at[pl.ds(g * 16 + j, 1)],
                        sem,
                    )
                )
            for cp in copies:
                cp.wait()
            return carry2

        lax.fori_loop(0, _RC // 16, group, 0)
        pltpu.sync_copy(rows_v, out_hbm.at[pl.ds(base + bc * _RC, _RC)])
        return carry

    lax.fori_loop(0, _BPW // _RC, big, 0)


def _sc_gather(table, idx2):
    mesh = plsc.VectorSubcoreMesh(core_axis_name="c", subcore_axis_name="s")
    return pl.kernel(
        _gather_body,
        mesh=mesh,
        out_type=jax.ShapeDtypeStruct((_NTOK, _DOUT), jnp.float32),
        scratch_types=[
            pltpu.VMEM((_BPW,), jnp.int32),
            pltpu.VMEM((_RC, _DOUT), jnp.float32),
            pltpu.SemaphoreType.DMA,
        ],
        compiler_params=pltpu.CompilerParams(needs_layout_passes=False),
    )(table, idx2)


_T = 1024


def _moe_body(x_ref, wg_ref, we_ref, sel_ref, bg_ref, o_ref):
    x = x_ref[...]
    logits = jnp.dot(x, wg_ref[...], preferred_element_type=jnp.float32)
    g = jax.nn.softmax(logits, axis=-1)
    y = jnp.dot(
        x.astype(jnp.bfloat16), we_ref[...], preferred_element_type=jnp.float32
    )
    grep = jnp.dot(g, sel_ref[...], preferred_element_type=jnp.float32)
    z = (y * grep).reshape(_T, 4, 128)
    s = jnp.sum(z, axis=1)
    acc = s[:, :_DOUT] + s[:, _DOUT:]
    acc = acc + jnp.dot(g, bg_ref[...], preferred_element_type=jnp.float32)
    o_ref[...] = acc


def _tc_moe(x, wg, we_cat, sel, be):
    return pl.pallas_call(
        _moe_body,
        grid=(_NTOK // _T,),
        in_specs=[
            pl.BlockSpec((_T, _DIN), lambda i: (i, 0)),
            pl.BlockSpec((_DIN, _E), lambda i: (0, 0)),
            pl.BlockSpec((_DIN, _E * _DOUT), lambda i: (0, 0)),
            pl.BlockSpec((_E, _E * _DOUT), lambda i: (0, 0)),
            pl.BlockSpec((_E, _DOUT), lambda i: (0, 0)),
        ],
        out_specs=pl.BlockSpec((_T, _DOUT), lambda i: (i, 0)),
        out_shape=jax.ShapeDtypeStruct((_NTOK, _DOUT), jnp.float32),
    )(x, wg, we_cat, sel, be)


_TA = 2048


def _add_body(a_ref, b_ref, o_ref):
    o_ref[...] = a_ref[...] + b_ref[...]


def _tc_add(a, b):
    return pl.pallas_call(
        _add_body,
        grid=(_NTOK // _TA,),
        in_specs=[
            pl.BlockSpec((_TA, _DOUT), lambda i: (i, 0)),
            pl.BlockSpec((_TA, _DOUT), lambda i: (i, 0)),
        ],
        out_specs=pl.BlockSpec((_TA, _DOUT), lambda i: (i, 0)),
        out_shape=jax.ShapeDtypeStruct((_NTOK, _DOUT), jnp.float32),
    )(a, b)


def kernel(item_feat, item_id, table, W_e, b_e, W_g):
    x = item_feat.transpose(1, 0, 2).reshape(_NTOK, _DIN)
    idx2 = item_id.T.reshape(_NW, _BPW).astype(jnp.int32)
    emb = _sc_gather(table, idx2)
    we_cat = jnp.transpose(W_e, (1, 0, 2)).reshape(_DIN, _E * _DOUT)
    we_cat = we_cat.astype(jnp.bfloat16)
    sel = jnp.asarray(np.repeat(np.eye(_E, dtype=np.float32), _DOUT, axis=1))
    moe = _tc_moe(x, W_g, we_cat, sel, b_e)
    out = _tc_add(moe, emb)
    return out.reshape(_L, _B, _DOUT).transpose(1, 0, 2)


# speedup vs baseline: 1.1100x; 1.0003x over previous
import functools

import jax
import jax.numpy as jnp
import numpy as np
from jax import lax
from jax.experimental import pallas as pl
from jax.experimental.pallas import tpu as pltpu
from jax.experimental.pallas import tpu_sc as plsc

_B, _L, _DIN, _DOUT, _E = 1024, 50, 768, 64, 8
_NTOK = _B * _L

_NC, _NS = 2, 16
_NW = _NC * _NS
_BPW = _NTOK // _NW
_C = 80
_K = _BPW // _C
_NBLK = 1000000 // 8


_RC = 160


def _gather_body(tab_hbm, idx_hbm, out_hbm, idx_v, rows_v, sem):
    wid = lax.axis_index("s") * _NC + lax.axis_index("c")
    base = wid * _BPW
    pltpu.sync_copy(idx_hbm.at[wid], idx_v)
    lanes = jnp.arange(16, dtype=jnp.int32)

    def big(bc, carry):
        def group(g, carry2):
            v16 = idx_v[pl.ds((bc * 10 + g) * 16, 16)]
            copies = []
            for j in range(16):
                i = jnp.sum(jnp.where(lanes == j, v16, 0))
                copies.append(
                    pltpu.async_copy(
                        tab_hbm.at[pl.ds(i, 1)],
                        rows_v.at[pl.ds(g * 16 + j, 1)],
                        sem,
                    )
                )
            for cp in copies:
                cp.wait()
            return carry2

        lax.fori_loop(0, _RC // 16, group, 0)
        pltpu.sync_copy(rows_v, out_hbm.at[pl.ds(base + bc * _RC, _RC)])
        return carry

    lax.fori_loop(0, _BPW // _RC, big, 0)


def _sc_gather(table, idx2):
    mesh = plsc.VectorSubcoreMesh(core_axis_name="c", subcore_axis_name="s")
    return pl.kernel(
        _gather_body,
        mesh=mesh,
        out_type=jax.ShapeDtypeStruct((_NTOK, _DOUT), jnp.float32),
        scratch_types=[
            pltpu.VMEM((_BPW,), jnp.int32),
            pltpu.VMEM((_RC, _DOUT), jnp.float32),
            pltpu.SemaphoreType.DMA,
        ],
        compiler_params=pltpu.CompilerParams(needs_layout_passes=False),
    )(table, idx2)


_T = 1024


def _moe_body(x_ref, wg_ref, we_ref, sel_ref, bg_ref, o_ref):
    x = x_ref[...]
    logits = jnp.dot(x, wg_ref[...], preferred_element_type=jnp.float32)
    g = jax.nn.softmax(logits, axis=-1)
    y = jnp.dot(
        x.astype(jnp.bfloat16), we_ref[...], preferred_element_type=jnp.float32
    )
    grep = jnp.dot(g, sel_ref[...], preferred_element_type=jnp.float32)
    z = (y * grep).reshape(_T, 4, 128)
    s = jnp.sum(z, axis=1)
    acc = s[:, :_DOUT] + s[:, _DOUT:]
    acc = acc + jnp.dot(g, bg_ref[...], preferred_element_type=jnp.float32)
    o_ref[...] = acc


def _tc_moe(x, wg, we_cat, sel, be):
    return pl.pallas_call(
        _moe_body,
        grid=(_NTOK // _T,),
        in_specs=[
            pl.BlockSpec((_T, _DIN), lambda i: (i, 0)),
            pl.BlockSpec((_DIN, _E), lambda i: (0, 0)),
            pl.BlockSpec((_DIN, _E * _DOUT), lambda i: (0, 0)),
            pl.BlockSpec((_E, _E * _DOUT), lambda i: (0, 0)),
            pl.BlockSpec((_E, _DOUT), lambda i: (0, 0)),
        ],
        out_specs=pl.BlockSpec((_T, _DOUT), lambda i: (i, 0)),
        out_shape=jax.ShapeDtypeStruct((_NTOK, _DOUT), jnp.float32),
    )(x, wg, we_cat, sel, be)


def _add_body(a_ref, b_ref, o_ref):
    o_ref[0] = jnp.transpose(a_ref[...] + b_ref[...])


def _tc_add(a, b):
    return pl.pallas_call(
        _add_body,
        grid=(_L,),
        in_specs=[
            pl.BlockSpec((_B, _DOUT), lambda i: (i, 0)),
            pl.BlockSpec((_B, _DOUT), lambda i: (i, 0)),
        ],
        out_specs=pl.BlockSpec((1, _DOUT, _B), lambda i: (i, 0, 0)),
        out_shape=jax.ShapeDtypeStruct((_L, _DOUT, _B), jnp.float32),
    )(a, b)


def kernel(item_feat, item_id, table, W_e, b_e, W_g):
    x = item_feat.transpose(1, 0, 2).reshape(_NTOK, _DIN)
    idx2 = item_id.T.reshape(_NW, _BPW).astype(jnp.int32)
    emb = _sc_gather(table, idx2)
    we_cat = jnp.transpose(W_e, (1, 0, 2)).reshape(_DIN, _E * _DOUT)
    we_cat = we_cat.astype(jnp.bfloat16)
    sel = jnp.asarray(np.repeat(np.eye(_E, dtype=np.float32), _DOUT, axis=1))
    moe = _tc_moe(x, W_g, we_cat, sel, b_e)
    out = _tc_add(moe, emb)
    return out.transpose(2, 0, 1)


# speedup vs baseline: 1.1180x; 1.0071x over previous
import functools

import jax
import jax.numpy as jnp
import numpy as np
from jax import lax
from jax.experimental import pallas as pl
from jax.experimental.pallas import tpu as pltpu
from jax.experimental.pallas import tpu_sc as plsc

_B, _L, _DIN, _DOUT, _E = 1024, 50, 768, 64, 8
_NTOK = _B * _L

_NC, _NS = 2, 16
_NW = _NC * _NS
_BPW = _NTOK // _NW
_C = 80
_K = _BPW // _C
_NBLK = 1000000 // 8


_RC = 160


def _gather_body(tab_hbm, idx_hbm, out_hbm, idx_v, rows_v, sem):
    wid = lax.axis_index("s") * _NC + lax.axis_index("c")
    base = wid * _BPW
    pltpu.sync_copy(idx_hbm.at[wid], idx_v)
    lanes = jnp.arange(16, dtype=jnp.int32)

    def big(bc, carry):
        def group(g, carry2):
            v16 = idx_v[pl.ds((bc * 10 + g) * 16, 16)]
            copies = []
            for j in range(16):
                i = jnp.sum(jnp.where(lanes == j, v16, 0))
                copies.append(
                    pltpu.async_copy(
                        tab_hbm.at[pl.ds(i, 1)],
                        rows_v.at[pl.ds(g * 16 + j, 1)],
                        sem,
                    )
                )
            for cp in copies:
                cp.wait()
            return carry2

        lax.fori_loop(0, _RC // 16, group, 0)
        pltpu.sync_copy(rows_v, out_hbm.at[pl.ds(base + bc * _RC, _RC)])
        return carry

    lax.fori_loop(0, _BPW // _RC, big, 0)


def _sc_gather(table, idx2):
    mesh = plsc.VectorSubcoreMesh(core_axis_name="c", subcore_axis_name="s")
    return pl.kernel(
        _gather_body,
        mesh=mesh,
        out_type=jax.ShapeDtypeStruct((_NTOK, _DOUT), jnp.float32),
        scratch_types=[
            pltpu.VMEM((_BPW,), jnp.int32),
            pltpu.VMEM((_RC, _DOUT), jnp.float32),
            pltpu.SemaphoreType.DMA,
        ],
        compiler_params=pltpu.CompilerParams(needs_layout_passes=False),
    )(table, idx2)


_T = 2048


def _moe_body(x_ref, wg_ref, we_ref, sel_ref, bg_ref, o_ref):
    x = x_ref[...]
    logits = jnp.dot(x, wg_ref[...], preferred_element_type=jnp.float32)
    g = jax.nn.softmax(logits, axis=-1)
    y = jnp.dot(
        x.astype(jnp.bfloat16), we_ref[...], preferred_element_type=jnp.float32
    )
    grep = jnp.dot(g, sel_ref[...], preferred_element_type=jnp.float32)
    z = (y * grep).reshape(_T, 4, 128)
    s = jnp.sum(z, axis=1)
    acc = s[:, :_DOUT] + s[:, _DOUT:]
    acc = acc + jnp.dot(g, bg_ref[...], preferred_element_type=jnp.float32)
    o_ref[...] = acc


def _tc_moe(x, wg, we_cat, sel, be):
    return pl.pallas_call(
        _moe_body,
        grid=(_NTOK // _T,),
        in_specs=[
            pl.BlockSpec((_T, _DIN), lambda i: (i, 0)),
            pl.BlockSpec((_DIN, _E), lambda i: (0, 0)),
            pl.BlockSpec((_DIN, _E * _DOUT), lambda i: (0, 0)),
            pl.BlockSpec((_E, _E * _DOUT), lambda i: (0, 0)),
            pl.BlockSpec((_E, _DOUT), lambda i: (0, 0)),
        ],
        out_specs=pl.BlockSpec((_T, _DOUT), lambda i: (i, 0)),
        out_shape=jax.ShapeDtypeStruct((_NTOK, _DOUT), jnp.float32),
    )(x, wg, we_cat, sel, be)


def _add_body(a_ref, b_ref, o_ref):
    o_ref[0] = jnp.transpose(a_ref[...] + b_ref[...])


def _tc_add(a, b):
    return pl.pallas_call(
        _add_body,
        grid=(_L,),
        in_specs=[
            pl.BlockSpec((_B, _DOUT), lambda i: (i, 0)),
            pl.BlockSpec((_B, _DOUT), lambda i: (i, 0)),
        ],
        out_specs=pl.BlockSpec((1, _DOUT, _B), lambda i: (i, 0, 0)),
        out_shape=jax.ShapeDtypeStruct((_L, _DOUT, _B), jnp.float32),
    )(a, b)


def kernel(item_feat, item_id, table, W_e, b_e, W_g):
    x = item_feat.transpose(1, 0, 2).reshape(_NTOK, _DIN)
    idx2 = item_id.T.reshape(_NW, _BPW).astype(jnp.int32)
    emb = _sc_gather(table, idx2)
    we_cat = jnp.transpose(W_e, (1, 0, 2)).reshape(_DIN, _E * _DOUT)
    we_cat = we_cat.astype(jnp.bfloat16)
    sel = jnp.asarray(np.repeat(np.eye(_E, dtype=np.float32), _DOUT, axis=1))
    moe = _tc_moe(x, W_g, we_cat, sel, b_e)
    out = _tc_add(moe, emb)
    return out.transpose(2, 0, 1)
